# baseline (speedup 1.0000x reference)
import functools

import jax
import jax.numpy as jnp
from jax.experimental import pallas as pl

_N = 50000
_B = 64
_D = 16
_R = 114
_RATIO = 0.8
_NEG = 0.2
_BE = 3200


def _edge_kernel(hs_ref, hd_ref, et_ref, w_ref, u_ref, v_ref,
                 alpha_ref, msg_ref):
    hs = hs_ref[...]
    hd = hd_ref[...]
    et = et_ref[...]
    iota = jax.lax.broadcasted_iota(jnp.int32, (et.shape[0], _R), 1)
    onehot = (et == iota).astype(jnp.float32)
    ue = jnp.dot(onehot, u_ref[...], preferred_element_type=jnp.float32)
    ve = jnp.dot(onehot, v_ref[...], preferred_element_type=jnp.float32)
    a = jnp.sum(hd * ue + hs * ve, axis=1, keepdims=True)
    alpha_ref[...] = jnp.where(a >= 0, a, _NEG * a)

    def body(r, acc):
        w = w_ref[r]
        m = (et[:, 0] == r).astype(jnp.float32)
        return acc + m[:, None] * jnp.dot(hs, w,
                                          preferred_element_type=jnp.float32)

    msg_ref[...] = jax.lax.fori_loop(
        0, _R, body, jnp.zeros((hs.shape[0], _D), jnp.float32))


@functools.partial(jax.jit, static_argnames=())
def _edge_pass(hs, hd, et2d, W, U, V):
    e = hs.shape[0]
    grid = e // _BE
    alpha, msg = pl.pallas_call(
        _edge_kernel,
        grid=(grid,),
        in_specs=[
            pl.BlockSpec((_BE, _D), lambda i: (i, 0)),
            pl.BlockSpec((_BE, _D), lambda i: (i, 0)),
            pl.BlockSpec((_BE, 1), lambda i: (i, 0)),
            pl.BlockSpec((_R, _D, _D), lambda i: (0, 0, 0)),
            pl.BlockSpec((_R, _D), lambda i: (0, 0)),
            pl.BlockSpec((_R, _D), lambda i: (0, 0)),
        ],
        out_specs=[
            pl.BlockSpec((_BE, 1), lambda i: (i, 0)),
            pl.BlockSpec((_BE, _D), lambda i: (i, 0)),
        ],
        out_shape=[
            jax.ShapeDtypeStruct((e, 1), jnp.float32),
            jax.ShapeDtypeStruct((e, _D), jnp.float32),
        ],
    )(hs, hd, et2d, W, U, V)
    return alpha[:, 0], msg


def _mlp_kernel(g_ref, w1_ref, b1_ref, w2_ref, b2_ref, w3_ref, b3_ref, o_ref):
    g = g_ref[...]
    a = jnp.maximum(
        jnp.dot(g, w1_ref[...], preferred_element_type=jnp.float32)
        + b1_ref[...], 0.0)
    a = jnp.maximum(
        jnp.dot(a, w2_ref[...], preferred_element_type=jnp.float32)
        + b2_ref[...], 0.0)
    a = jnp.dot(a, w3_ref[...], preferred_element_type=jnp.float32) + b3_ref[...]
    o_ref[...] = jax.nn.sigmoid(a)


def _mlp(g, l1w, l1b, l2w, l2b, l3w, l3b):
    return pl.pallas_call(
        _mlp_kernel,
        out_shape=jax.ShapeDtypeStruct((g.shape[0], 1), jnp.float32),
    )(g, l1w.T, l1b[None, :], l2w.T, l2b[None, :], l3w.T, l3b[None, :])


def _rgat_layer(h, src, dst, et2d, emask, W, q, k, b):
    n = h.shape[0]
    U = jnp.einsum('rdo,o->rd', W, q[:, 0])
    V = jnp.einsum('rdo,o->rd', W, k[:, 0])
    hs = h[src]
    hd = h[dst]
    alpha, msg = _edge_pass(hs, hd, et2d, W, U, V)
    alpha = jnp.where(emask, alpha, -jnp.inf)
    seg_max = jax.ops.segment_max(alpha, dst, num_segments=n)
    seg_max = jnp.where(jnp.isfinite(seg_max), seg_max, 0.0)
    ex = jnp.exp(alpha - seg_max[dst])
    ex = jnp.where(emask, ex, 0.0)
    denom = jax.ops.segment_sum(ex, dst, num_segments=n)
    att = ex / (denom[dst] + 1e-16)
    out = jax.ops.segment_sum(att[:, None] * msg, dst, num_segments=n)
    return out + b


def _pool_stats(h, batch, nmask):
    m = nmask.astype(h.dtype)[:, None]
    s = jax.ops.segment_sum(h * m, batch, num_segments=_B)
    cnt = jax.ops.segment_sum(m[:, 0], batch, num_segments=_B)
    mean = s / jnp.maximum(cnt, 1.0)[:, None]
    hm = jnp.where(nmask[:, None], h, -jnp.inf)
    mx = jax.ops.segment_max(hm, batch, num_segments=_B)
    mx = jnp.where(jnp.isfinite(mx), mx, 0.0)
    return mean, mx


def _topk_pool(h, w, batch, nmask):
    n = h.shape[0]
    score = jnp.tanh((h @ w) / jnp.linalg.norm(w))
    counts_all = jnp.bincount(batch, length=_B)
    starts = jnp.concatenate(
        [jnp.zeros((1,), counts_all.dtype), jnp.cumsum(counts_all)[:-1]])
    alive_counts = jax.ops.segment_sum(
        nmask.astype(jnp.int32), batch, num_segments=_B)
    key1 = jnp.where(nmask, -score, jnp.inf)
    ord1 = jnp.argsort(key1)
    order = ord1[jnp.argsort(batch[ord1])]
    ranks_sorted = jnp.arange(n) - starts[batch[order]]
    rank = jnp.zeros((n,), ranks_sorted.dtype).at[order].set(ranks_sorted)
    kk = jnp.ceil(_RATIO * alive_counts.astype(jnp.float32)).astype(rank.dtype)
    new_mask = nmask & (rank < kk[batch])
    h_new = h * score[:, None] * new_mask.astype(h.dtype)[:, None]
    return h_new, new_mask


def kernel(x, edge_index, edge_attr, seq, emb, W1, q1, k1, b1, W2, q2, k2, b2,
           W3, q3, k3, b3, pw1, pw2, l1w, l1b, l2w, l2b, l3w, l3b):
    h = emb[x]
    src = edge_index[0]
    dst = edge_index[1]
    e = src.shape[0]
    epad = ((e + _BE - 1) // _BE) * _BE
    pad = epad - e
    et2d = jnp.pad(edge_attr.astype(jnp.int32), (0, pad))[:, None]
    src_p = jnp.pad(src, (0, pad))
    dst_p = jnp.pad(dst, (0, pad))
    batch = seq
    nmask = jnp.ones((h.shape[0],), dtype=bool)
    emask = jnp.pad(jnp.ones((e,), dtype=bool), (0, pad))
    layers = [(W1, q1, k1, b1), (W2, q2, k2, b2), (W3, q3, k3, b3)]
    pws = [pw1, pw2]
    feats = []
    for i in range(3):
        Wl, ql, kl, bl = layers[i]
        h = jax.nn.relu(_rgat_layer(h, src_p, dst_p, et2d, emask, Wl, ql, kl, bl))
        mean, mx = _pool_stats(h, batch, nmask)
        feats.append(mean)
        feats.append(mx)
        if i < 2:
            h, nmask = _topk_pool(h, pws[i], batch, nmask)
            emask = emask & nmask[src_p] & nmask[dst_p]
    g = jnp.concatenate(feats, axis=1)
    return _mlp(g, l1w, l1b, l2w, l2b, l3w, l3b)
